# baseline (speedup 1.0000x reference)
import functools

import jax
import jax.numpy as jnp
from jax.experimental import pallas as pl
from jax.experimental.pallas import tpu as pltpu

_BLK = 5000
_EPS = 1e-5
_SLOPE = 0.2


def _stage_body(*refs, n_total, nblk, blk, k3, gn, resx, final, emit_stats):
    it = iter(refs)
    h_ref = next(it)
    if gn:
        stats_ref = next(it)
        bprev_ref = next(it)
        gamma_ref = next(it)
        beta_ref = next(it)
        alpha_ref = next(it)
    a_ref = next(it)
    b_ref = next(it)
    c_ref = next(it) if k3 else None
    bias_ref = next(it) if (resx or final) else None
    x_ref = next(it) if resx else None
    if final:
        linw_ref = next(it)
        linb_ref = next(it)
        fin_ref = next(it)
    else:
        out_ref = next(it)
        so_ref = next(it) if emit_stats else None
    carry_ref = next(it)
    sums_ref = next(it) if final else None

    i = pl.program_id(0)
    if gn:
        s = stats_ref[...]
        inv_n = 1.0 / n_total
        b = bprev_ref[...]
        mu0 = s[0:1, :] * inv_n
        mu = mu0 + b
        msq = s[1:2, :] * inv_n + b * (2.0 * mu0 + b)
        am = alpha_ref[...] * mu
        var = msq - 2.0 * am * mu + am * am
        g = gamma_ref[...] * jax.lax.rsqrt(var + _EPS)
        cadd = beta_ref[...] - g * am + g * b
        gb = g.astype(jnp.bfloat16)
        cb = cadd.astype(jnp.bfloat16)
        t = h_ref[...] * gb + cb
        act = jnp.maximum(t, jnp.bfloat16(_SLOPE) * t)
    else:
        act = h_ref[...].astype(jnp.bfloat16)

    @pl.when(i == 0)
    def _():
        carry_ref[...] = jnp.zeros_like(carry_ref)

    prev = carry_ref[0:2, :]
    sh1 = jnp.concatenate([prev[1:2, :], act[: blk - 1, :]], axis=0)
    if k3:
        sh2 = jnp.concatenate([prev[0:2, :], act[: blk - 2, :]], axis=0)
    carry_ref[0:2, :] = act[blk - 2 :, :]

    out = jnp.dot(act, a_ref[...], preferred_element_type=jnp.float32)
    out = out + jnp.dot(sh1, b_ref[...], preferred_element_type=jnp.float32)
    if k3:
        out = out + jnp.dot(sh2, c_ref[...], preferred_element_type=jnp.float32)

    if resx:
        ob = out.astype(jnp.bfloat16)
        ob = jnp.maximum((ob + x_ref[...]) + bias_ref[...],
                         jnp.zeros((), jnp.bfloat16))
    else:
        ob = out.astype(jnp.bfloat16)

    def _last_row_fix():
        delta = jnp.dot(act[blk - 2 : blk - 1, :], b_ref[...],
                        preferred_element_type=jnp.float32)
        if k3:
            delta = delta + jnp.dot(act[blk - 3 : blk - 2, :], c_ref[...],
                                    preferred_element_type=jnp.float32)
        fr = out[blk - 1 : blk, :] - delta
        frb = fr.astype(jnp.bfloat16)
        if resx:
            frb = jnp.maximum((frb + x_ref[blk - 1 : blk, :]) + bias_ref[...],
                              jnp.zeros((), jnp.bfloat16))
        return frb

    if final:
        @pl.when(i == 0)
        def _():
            sums_ref[...] = jnp.zeros_like(sums_ref)

        sums_ref[0:1, :] = sums_ref[0:1, :] + jnp.sum(
            ob.astype(jnp.float32), axis=0, keepdims=True)

        @pl.when(i == nblk - 1)
        def _():
            frb = _last_row_fix()
            wrongb = ob[blk - 1 : blk, :]
            corr = frb.astype(jnp.float32) - wrongb.astype(jnp.float32)
            pooled = (sums_ref[0:1, :] + corr) * (1.0 / n_total)
            o = jnp.dot(pooled, linw_ref[...], preferred_element_type=jnp.float32)
            fin_ref[...] = jnp.tanh(o + linb_ref[...])
    else:
        out_ref[...] = ob
        if emit_stats:
            s0 = jnp.sum(out, axis=0, keepdims=True)
            s1 = jnp.sum(out * out, axis=0, keepdims=True)
            new = jnp.concatenate([s0, s1], axis=0)
            so_ref[...] = jnp.where(i == 0, new, so_ref[...] + new)

        @pl.when(i == nblk - 1)
        def _():
            frb = _last_row_fix()
            out_ref[blk - 1 : blk, :] = frb
            wrong = out[blk - 1 : blk, :]
            if emit_stats:
                fr32 = frb.astype(jnp.float32)
                d0 = fr32 - wrong
                d1 = fr32 * fr32 - wrong * wrong
                so_ref[...] = so_ref[...] + jnp.concatenate([d0, d1], axis=0)


def _run_stage(h, stats, bprev, gnp, Ws, bias, xres, lin, *, emit_stats, final):
    n_total, din = h.shape
    dout = Ws[0].shape[1]
    k3 = len(Ws) == 3
    blk = _BLK if n_total % _BLK == 0 else n_total
    nblk = n_total // blk

    if k3:
        wa, wb, wc = Ws[0] - Ws[2], -Ws[1], 2.0 * Ws[2]
    else:
        wa, wb, wc = Ws[0], -Ws[1], None
    wa, wb = wa.astype(jnp.bfloat16), wb.astype(jnp.bfloat16)
    wc = wc.astype(jnp.bfloat16) if k3 else None

    const = lambda shape: pl.BlockSpec(shape, lambda i: (0, 0))
    rows = lambda width: pl.BlockSpec((blk, width), lambda i: (i, 0))

    inputs = [h]
    in_specs = [rows(din)]
    gn = stats is not None
    if gn:
        gamma, beta, alpha = gnp
        inputs += [stats, bprev.reshape(1, din), gamma.reshape(1, din),
                   beta.reshape(1, din), alpha.reshape(1, din)]
        in_specs += [const((2, din))] + [const((1, din))] * 4
    inputs += [wa, wb] + ([wc] if k3 else [])
    in_specs += [const((din, dout))] * (3 if k3 else 2)
    resx = xres is not None
    if resx or final:
        inputs.append(bias.reshape(1, dout).astype(jnp.bfloat16)
                      if resx else bias.reshape(1, dout))
    if resx:
        in_specs.append(const((1, dout)))
        inputs.append(xres)
        in_specs.append(rows(xres.shape[1]))
    scratch = [pltpu.VMEM((16, din), jnp.bfloat16)]
    if final:
        linw, linb = lin
        inputs += [linw, linb]
        in_specs += [const(linw.shape), const((1, linb.shape[-1]))]
        out_shape = jax.ShapeDtypeStruct((1, linb.shape[-1]), jnp.float32)
        out_specs = const((1, linb.shape[-1]))
        scratch.append(pltpu.VMEM((8, dout), jnp.float32))
    elif emit_stats:
        out_shape = (jax.ShapeDtypeStruct((n_total, dout), jnp.bfloat16),
                     jax.ShapeDtypeStruct((2, dout), jnp.float32))
        out_specs = (rows(dout), const((2, dout)))
    else:
        out_shape = jax.ShapeDtypeStruct((n_total, dout), jnp.bfloat16)
        out_specs = rows(dout)

    body = functools.partial(
        _stage_body, n_total=n_total, nblk=nblk, blk=blk, k3=k3, gn=gn,
        resx=resx, final=final, emit_stats=emit_stats)
    return pl.pallas_call(
        body,
        grid=(nblk,),
        in_specs=in_specs,
        out_specs=out_specs,
        out_shape=out_shape,
        scratch_shapes=scratch,
        compiler_params=pltpu.CompilerParams(
            dimension_semantics=("arbitrary",)),
    )(*inputs)


def kernel(x, params):
    convs = params["convs"]
    gns = params["gns"]
    lin = (params["lin_W"].T, params["lin_b"].reshape(1, -1))
    x_res = x.astype(jnp.bfloat16)
    cur = x
    h = None
    stats = None
    for blk_i in range(4):
        for j in range(4):
            ci = 4 * blk_i + j
            final = ci == 15
            gnp = None
            if j > 0:
                g = gns[3 * blk_i + (j - 1)]
                gnp = (g["gamma"], g["beta"], g["alpha"])
            res = _run_stage(
                cur if j == 0 else h,
                stats if j > 0 else None,
                convs[ci - 1]["b"] if j > 0 else None,
                gnp,
                convs[ci]["Ws"],
                convs[ci]["b"],
                x_res if j == 3 else None,
                lin if final else None,
                emit_stats=j < 3,
                final=final,
            )
            if final:
                return res
            if j < 3:
                h, stats = res
            else:
                cur = res


# speedup vs baseline: 1.0007x; 1.0007x over previous
import functools

import jax
import jax.numpy as jnp
from jax.experimental import pallas as pl
from jax.experimental.pallas import tpu as pltpu

_BLK = 10000
_EPS = 1e-5
_SLOPE = 0.2


def _gn_prologue(h_ref, stats_ref, bprev_ref, gamma_ref, beta_ref, alpha_ref,
                 n_total):
    s = stats_ref[...]
    inv_n = 1.0 / n_total
    b = bprev_ref[...]
    mu0 = s[0:1, :] * inv_n
    mu = mu0 + b
    msq = s[1:2, :] * inv_n + b * (2.0 * mu0 + b)
    am = alpha_ref[...] * mu
    var = msq - 2.0 * am * mu + am * am
    g = gamma_ref[...] * jax.lax.rsqrt(var + _EPS)
    cadd = beta_ref[...] - g * am + g * b
    t = h_ref[...] * g.astype(jnp.bfloat16) + cadd.astype(jnp.bfloat16)
    return jnp.maximum(t, jnp.bfloat16(_SLOPE) * t)


def _shift1(prev2, act, blk):
    return jnp.concatenate([prev2[1:2, :], act[: blk - 1, :]], axis=0)


def _stage_body(*refs, n_total, nblk, blk, k3, gn, resx, final, emit_stats,
                emit_xbf):
    it = iter(refs)
    h_ref = next(it)
    if gn:
        stats_ref = next(it)
        bprev_ref = next(it)
        gamma_ref = next(it)
        beta_ref = next(it)
        alpha_ref = next(it)
    a_ref = next(it)
    b_ref = next(it)
    c_ref = next(it) if k3 else None
    bias_ref = next(it) if (resx or final) else None
    x_ref = next(it) if resx else None
    if final:
        linw_ref = next(it)
        linb_ref = next(it)
        fin_ref = next(it)
    else:
        out_ref = next(it)
        so_ref = next(it) if emit_stats else None
        xb_ref = next(it) if emit_xbf else None
    carry_ref = next(it)
    sums_ref = next(it) if final else None

    i = pl.program_id(0)
    if gn:
        act = _gn_prologue(h_ref, stats_ref, bprev_ref, gamma_ref, beta_ref,
                           alpha_ref, n_total)
    else:
        act = h_ref[...].astype(jnp.bfloat16)
    if emit_xbf:
        xb_ref[...] = act

    @pl.when(i == 0)
    def _():
        carry_ref[...] = jnp.zeros_like(carry_ref)

    prev = carry_ref[0:2, :]
    sh1 = _shift1(prev, act, blk)
    if k3:
        sh2 = jnp.concatenate([prev[0:2, :], act[: blk - 2, :]], axis=0)
    carry_ref[0:2, :] = act[blk - 2 :, :]

    out = jnp.dot(act, a_ref[...], preferred_element_type=jnp.float32)
    out = out + jnp.dot(sh1, b_ref[...], preferred_element_type=jnp.float32)
    if k3:
        out = out + jnp.dot(sh2, c_ref[...], preferred_element_type=jnp.float32)

    if resx:
        ob = out.astype(jnp.bfloat16)
        ob = jnp.maximum((ob + x_ref[...]) + bias_ref[...],
                         jnp.zeros((), jnp.bfloat16))
    else:
        ob = out.astype(jnp.bfloat16)

    def _last_row_fix():
        delta = jnp.dot(act[blk - 2 : blk - 1, :], b_ref[...],
                        preferred_element_type=jnp.float32)
        if k3:
            delta = delta + jnp.dot(act[blk - 3 : blk - 2, :], c_ref[...],
                                    preferred_element_type=jnp.float32)
        fr = out[blk - 1 : blk, :] - delta
        frb = fr.astype(jnp.bfloat16)
        if resx:
            frb = jnp.maximum((frb + x_ref[blk - 1 : blk, :]) + bias_ref[...],
                              jnp.zeros((), jnp.bfloat16))
        return frb

    if final:
        @pl.when(i == 0)
        def _():
            sums_ref[...] = jnp.zeros_like(sums_ref)

        sums_ref[0:1, :] = sums_ref[0:1, :] + jnp.sum(
            ob.astype(jnp.float32), axis=0, keepdims=True)

        @pl.when(i == nblk - 1)
        def _():
            frb = _last_row_fix()
            wrongb = ob[blk - 1 : blk, :]
            corr = frb.astype(jnp.float32) - wrongb.astype(jnp.float32)
            pooled = (sums_ref[0:1, :] + corr) * (1.0 / n_total)
            o = jnp.dot(pooled, linw_ref[...], preferred_element_type=jnp.float32)
            fin_ref[...] = jnp.tanh(o + linb_ref[...])
    else:
        out_ref[...] = ob
        if emit_stats:
            s0 = jnp.sum(out, axis=0, keepdims=True)
            s1 = jnp.sum(out * out, axis=0, keepdims=True)
            new = jnp.concatenate([s0, s1], axis=0)
            so_ref[...] = jnp.where(i == 0, new, so_ref[...] + new)

        @pl.when(i == nblk - 1)
        def _():
            frb = _last_row_fix()
            out_ref[blk - 1 : blk, :] = frb
            if emit_stats:
                wrong = out[blk - 1 : blk, :]
                fr32 = frb.astype(jnp.float32)
                d0 = fr32 - wrong
                d1 = fr32 * fr32 - wrong * wrong
                so_ref[...] = so_ref[...] + jnp.concatenate([d0, d1], axis=0)


def _merged_body(*refs, n_total, nblk, blk):
    (h_ref, stats_ref, bprev_ref, gamma_ref, beta_ref, alpha_ref,
     a3_ref, b3_ref, bias3_ref, x_ref, a0_ref, b0_ref,
     out_ref, so_ref, carry3_ref, carry0_ref) = refs

    i = pl.program_id(0)
    act = _gn_prologue(h_ref, stats_ref, bprev_ref, gamma_ref, beta_ref,
                       alpha_ref, n_total)

    @pl.when(i == 0)
    def _():
        carry3_ref[...] = jnp.zeros_like(carry3_ref)
        carry0_ref[...] = jnp.zeros_like(carry0_ref)

    prev3 = carry3_ref[0:2, :]
    sh1 = _shift1(prev3, act, blk)
    carry3_ref[0:2, :] = act[blk - 2 :, :]
    out3 = jnp.dot(act, a3_ref[...], preferred_element_type=jnp.float32)
    out3 = out3 + jnp.dot(sh1, b3_ref[...], preferred_element_type=jnp.float32)
    cur = jnp.maximum((out3.astype(jnp.bfloat16) + x_ref[...]) + bias3_ref[...],
                      jnp.zeros((), jnp.bfloat16))

    prev0 = carry0_ref[0:2, :]
    sh1b = _shift1(prev0, cur, blk)
    carry0_ref[0:2, :] = cur[blk - 2 :, :]
    out0 = jnp.dot(cur, a0_ref[...], preferred_element_type=jnp.float32)
    out0 = out0 + jnp.dot(sh1b, b0_ref[...], preferred_element_type=jnp.float32)

    out_ref[...] = out0.astype(jnp.bfloat16)
    s0 = jnp.sum(out0, axis=0, keepdims=True)
    s1 = jnp.sum(out0 * out0, axis=0, keepdims=True)
    new = jnp.concatenate([s0, s1], axis=0)
    so_ref[...] = jnp.where(i == 0, new, so_ref[...] + new)

    @pl.when(i == nblk - 1)
    def _():
        delta3 = jnp.dot(act[blk - 2 : blk - 1, :], b3_ref[...],
                         preferred_element_type=jnp.float32)
        fr3 = out3[blk - 1 : blk, :] - delta3
        curfix = jnp.maximum(
            (fr3.astype(jnp.bfloat16) + x_ref[blk - 1 : blk, :]) + bias3_ref[...],
            jnp.zeros((), jnp.bfloat16))
        fr0 = jnp.dot(curfix, a0_ref[...], preferred_element_type=jnp.float32)
        out_ref[blk - 1 : blk, :] = fr0.astype(jnp.bfloat16)
        wrong = out0[blk - 1 : blk, :]
        d0 = fr0 - wrong
        d1 = fr0 * fr0 - wrong * wrong
        so_ref[...] = so_ref[...] + jnp.concatenate([d0, d1], axis=0)


def _w2(Ws):
    if len(Ws) == 3:
        return ((Ws[0] - Ws[2]).astype(jnp.bfloat16),
                (-Ws[1]).astype(jnp.bfloat16),
                (2.0 * Ws[2]).astype(jnp.bfloat16))
    return Ws[0].astype(jnp.bfloat16), (-Ws[1]).astype(jnp.bfloat16), None


def _gn_inputs(stats, bprev, gnp, din):
    gamma, beta, alpha = gnp
    return [stats, bprev.reshape(1, din), gamma.reshape(1, din),
            beta.reshape(1, din), alpha.reshape(1, din)]


def _blkshape(n_total):
    blk = _BLK if n_total % _BLK == 0 else n_total
    return blk, n_total // blk


_CP = pltpu.CompilerParams(dimension_semantics=("arbitrary",))


def _run_stage(h, stats, bprev, gnp, Ws, bias, xres, lin, *, emit_stats,
               final, emit_xbf=False):
    n_total, din = h.shape
    dout = Ws[0].shape[1]
    k3 = len(Ws) == 3
    blk, nblk = _blkshape(n_total)
    wa, wb, wc = _w2(Ws)

    const = lambda shape: pl.BlockSpec(shape, lambda i: (0, 0))
    rows = lambda width: pl.BlockSpec((blk, width), lambda i: (i, 0))

    inputs = [h]
    in_specs = [rows(din)]
    gn = stats is not None
    if gn:
        inputs += _gn_inputs(stats, bprev, gnp, din)
        in_specs += [const((2, din))] + [const((1, din))] * 4
    inputs += [wa, wb] + ([wc] if k3 else [])
    in_specs += [const((din, dout))] * (3 if k3 else 2)
    resx = xres is not None
    if resx or final:
        inputs.append(bias.reshape(1, dout).astype(jnp.bfloat16))
        in_specs.append(const((1, dout)))
    if resx:
        inputs.append(xres)
        in_specs.append(rows(xres.shape[1]))
    scratch = [pltpu.VMEM((16, din), jnp.bfloat16)]
    if final:
        linw, linb = lin
        inputs += [linw, linb]
        in_specs += [const(linw.shape), const((1, linb.shape[-1]))]
        out_shape = jax.ShapeDtypeStruct((1, linb.shape[-1]), jnp.float32)
        out_specs = const((1, linb.shape[-1]))
        scratch.append(pltpu.VMEM((8, dout), jnp.float32))
    else:
        out_shape = [jax.ShapeDtypeStruct((n_total, dout), jnp.bfloat16)]
        out_specs = [rows(dout)]
        if emit_stats:
            out_shape.append(jax.ShapeDtypeStruct((2, dout), jnp.float32))
            out_specs.append(const((2, dout)))
        if emit_xbf:
            out_shape.append(jax.ShapeDtypeStruct((n_total, din), jnp.bfloat16))
            out_specs.append(rows(din))
        out_shape = tuple(out_shape)
        out_specs = tuple(out_specs)

    body = functools.partial(
        _stage_body, n_total=n_total, nblk=nblk, blk=blk, k3=k3, gn=gn,
        resx=resx, final=final, emit_stats=emit_stats, emit_xbf=emit_xbf)
    return pl.pallas_call(
        body, grid=(nblk,), in_specs=in_specs, out_specs=out_specs,
        out_shape=out_shape, scratch_shapes=scratch, compiler_params=_CP,
    )(*inputs)


def _run_merged(h, stats, bprev, gnp, Ws3, bias3, xres, Ws0):
    n_total, din = h.shape
    dmid = Ws3[0].shape[1]
    dout = Ws0[0].shape[1]
    blk, nblk = _blkshape(n_total)
    wa3, wb3, _ = _w2(Ws3)
    wa0, wb0, _ = _w2(Ws0)

    const = lambda shape: pl.BlockSpec(shape, lambda i: (0, 0))
    rows = lambda width: pl.BlockSpec((blk, width), lambda i: (i, 0))

    inputs = ([h] + _gn_inputs(stats, bprev, gnp, din) +
              [wa3, wb3, bias3.reshape(1, dmid).astype(jnp.bfloat16), xres,
               wa0, wb0])
    in_specs = ([rows(din), const((2, din))] + [const((1, din))] * 4 +
                [const((din, dmid))] * 2 + [const((1, dmid)), rows(dmid)] +
                [const((dmid, dout))] * 2)
    out_shape = (jax.ShapeDtypeStruct((n_total, dout), jnp.bfloat16),
                 jax.ShapeDtypeStruct((2, dout), jnp.float32))
    out_specs = (rows(dout), const((2, dout)))
    scratch = [pltpu.VMEM((16, din), jnp.bfloat16),
               pltpu.VMEM((16, dmid), jnp.bfloat16)]
    body = functools.partial(_merged_body, n_total=n_total, nblk=nblk, blk=blk)
    return pl.pallas_call(
        body, grid=(nblk,), in_specs=in_specs, out_specs=out_specs,
        out_shape=out_shape, scratch_shapes=scratch, compiler_params=_CP,
    )(*inputs)


def kernel(x, params):
    convs = params["convs"]
    gns = params["gns"]
    lin = (params["lin_W"].T, params["lin_b"].reshape(1, -1))

    h, stats, x_res = _run_stage(
        x, None, None, None, convs[0]["Ws"], convs[0]["b"], None, None,
        emit_stats=True, final=False, emit_xbf=True)

    for blk_i in range(4):
        for j in (1, 2):
            ci = 4 * blk_i + j
            g = gns[3 * blk_i + (j - 1)]
            gnp = (g["gamma"], g["beta"], g["alpha"])
            h, stats = _run_stage(
                h, stats, convs[ci - 1]["b"], gnp, convs[ci]["Ws"],
                convs[ci]["b"], None, None, emit_stats=True, final=False)
        ci = 4 * blk_i + 3
        g = gns[3 * blk_i + 2]
        gnp = (g["gamma"], g["beta"], g["alpha"])
        if blk_i < 3:
            h, stats = _run_merged(
                h, stats, convs[ci - 1]["b"], gnp, convs[ci]["Ws"],
                convs[ci]["b"], x_res, convs[ci + 1]["Ws"])
        else:
            return _run_stage(
                h, stats, convs[ci - 1]["b"], gnp, convs[ci]["Ws"],
                convs[ci]["b"], x_res, lin, emit_stats=False, final=True)


# speedup vs baseline: 1.0624x; 1.0616x over previous
import functools

import jax
import jax.numpy as jnp
from jax.experimental import pallas as pl
from jax.experimental.pallas import tpu as pltpu

_BLK = 10000
_EPS = 1e-5
_SLOPE = 0.2


def _gn_prologue(h_ref, stats_ref, bprev_ref, gamma_ref, beta_ref, alpha_ref,
                 n_total):
    s = stats_ref[...]
    inv_n = 1.0 / n_total
    b = bprev_ref[...]
    mu0 = s[0:1, :] * inv_n
    mu = mu0 + b
    msq = s[1:2, :] * inv_n + b * (2.0 * mu0 + b)
    am = alpha_ref[...] * mu
    var = msq - 2.0 * am * mu + am * am
    g = gamma_ref[...] * jax.lax.rsqrt(var + _EPS)
    cadd = beta_ref[...] - g * am + g * b
    t = h_ref[...] * g.astype(jnp.bfloat16) + cadd.astype(jnp.bfloat16)
    return jnp.maximum(t, jnp.bfloat16(_SLOPE) * t)


def _shift1(prev2, act, blk):
    return jnp.concatenate([prev2[1:2, :], act[: blk - 1, :]], axis=0)


def _stage_body(*refs, n_total, nblk, blk, k3, gn, resx, final, emit_stats,
                emit_xbf):
    it = iter(refs)
    h_ref = next(it)
    if gn:
        stats_ref = next(it)
        bprev_ref = next(it)
        gamma_ref = next(it)
        beta_ref = next(it)
        alpha_ref = next(it)
    a_ref = next(it)
    b_ref = next(it)
    c_ref = next(it) if k3 else None
    bias_ref = next(it) if (resx or final) else None
    x_ref = next(it) if resx else None
    if final:
        linw_ref = next(it)
        linb_ref = next(it)
        fin_ref = next(it)
    else:
        out_ref = next(it)
        so_ref = next(it) if emit_stats else None
        xb_ref = next(it) if emit_xbf else None
    carry_ref = next(it)
    sums_ref = next(it) if final else None

    i = pl.program_id(0)
    if gn:
        act = _gn_prologue(h_ref, stats_ref, bprev_ref, gamma_ref, beta_ref,
                           alpha_ref, n_total)
    else:
        act = h_ref[...].astype(jnp.bfloat16)
    if emit_xbf:
        xb_ref[...] = act

    @pl.when(i == 0)
    def _():
        carry_ref[...] = jnp.zeros_like(carry_ref)

    prev = carry_ref[0:2, :]
    sh1 = _shift1(prev, act, blk)
    if k3:
        sh2 = jnp.concatenate([prev[0:2, :], act[: blk - 2, :]], axis=0)
    carry_ref[0:2, :] = act[blk - 2 :, :]

    out = jnp.dot(act, a_ref[...], preferred_element_type=jnp.float32)
    out = out + jnp.dot(sh1, b_ref[...], preferred_element_type=jnp.float32)
    if k3:
        out = out + jnp.dot(sh2, c_ref[...], preferred_element_type=jnp.float32)

    if resx:
        ob = out.astype(jnp.bfloat16)
        ob = jnp.maximum((ob + x_ref[...]) + bias_ref[...],
                         jnp.zeros((), jnp.bfloat16))
    else:
        ob = out.astype(jnp.bfloat16)

    def _last_row_fix():
        delta = jnp.dot(act[blk - 2 : blk - 1, :], b_ref[...],
                        preferred_element_type=jnp.float32)
        if k3:
            delta = delta + jnp.dot(act[blk - 3 : blk - 2, :], c_ref[...],
                                    preferred_element_type=jnp.float32)
        fr = out[blk - 1 : blk, :] - delta
        frb = fr.astype(jnp.bfloat16)
        if resx:
            frb = jnp.maximum((frb + x_ref[blk - 1 : blk, :]) + bias_ref[...],
                              jnp.zeros((), jnp.bfloat16))
        return frb

    if final:
        @pl.when(i == 0)
        def _():
            sums_ref[...] = jnp.zeros_like(sums_ref)

        sums_ref[0:1, :] = sums_ref[0:1, :] + jnp.sum(
            ob.astype(jnp.float32), axis=0, keepdims=True)

        @pl.when(i == nblk - 1)
        def _():
            frb = _last_row_fix()
            wrongb = ob[blk - 1 : blk, :]
            corr = frb.astype(jnp.float32) - wrongb.astype(jnp.float32)
            pooled = (sums_ref[0:1, :] + corr) * (1.0 / n_total)
            o = jnp.dot(pooled, linw_ref[...], preferred_element_type=jnp.float32)
            fin_ref[...] = jnp.tanh(o + linb_ref[...])
    else:
        out_ref[...] = ob
        if emit_stats:
            s0 = jnp.sum(out, axis=0, keepdims=True)
            s1 = jnp.sum(out * out, axis=0, keepdims=True)
            new = jnp.concatenate([s0, s1], axis=0)
            so_ref[...] = jnp.where(i == 0, new, so_ref[...] + new)

        @pl.when(i == nblk - 1)
        def _():
            frb = _last_row_fix()
            out_ref[blk - 1 : blk, :] = frb
            if emit_stats:
                wrong = out[blk - 1 : blk, :]
                fr32 = frb.astype(jnp.float32)
                d0 = fr32 - wrong
                d1 = fr32 * fr32 - wrong * wrong
                so_ref[...] = so_ref[...] + jnp.concatenate([d0, d1], axis=0)


def _merged_body(*refs, n_total, nblk, blk):
    (h_ref, stats_ref, bprev_ref, gamma_ref, beta_ref, alpha_ref,
     a3_ref, b3_ref, bias3_ref, x_ref, a0_ref, b0_ref,
     out_ref, so_ref, carry3_ref, carry0_ref) = refs

    i = pl.program_id(0)
    act = _gn_prologue(h_ref, stats_ref, bprev_ref, gamma_ref, beta_ref,
                       alpha_ref, n_total)

    @pl.when(i == 0)
    def _():
        carry3_ref[...] = jnp.zeros_like(carry3_ref)
        carry0_ref[...] = jnp.zeros_like(carry0_ref)

    prev3 = carry3_ref[0:2, :]
    sh1 = _shift1(prev3, act, blk)
    carry3_ref[0:2, :] = act[blk - 2 :, :]
    out3 = jnp.dot(act, a3_ref[...], preferred_element_type=jnp.float32)
    out3 = out3 + jnp.dot(sh1, b3_ref[...], preferred_element_type=jnp.float32)
    cur = jnp.maximum((out3.astype(jnp.bfloat16) + x_ref[...]) + bias3_ref[...],
                      jnp.zeros((), jnp.bfloat16))

    prev0 = carry0_ref[0:2, :]
    sh1b = _shift1(prev0, cur, blk)
    carry0_ref[0:2, :] = cur[blk - 2 :, :]
    out0 = jnp.dot(cur, a0_ref[...], preferred_element_type=jnp.float32)
    out0 = out0 + jnp.dot(sh1b, b0_ref[...], preferred_element_type=jnp.float32)

    out_ref[...] = out0.astype(jnp.bfloat16)
    s0 = jnp.sum(out0, axis=0, keepdims=True)
    s1 = jnp.sum(out0 * out0, axis=0, keepdims=True)
    new = jnp.concatenate([s0, s1], axis=0)
    so_ref[...] = jnp.where(i == 0, new, so_ref[...] + new)

    @pl.when(i == nblk - 1)
    def _():
        delta3 = jnp.dot(act[blk - 2 : blk - 1, :], b3_ref[...],
                         preferred_element_type=jnp.float32)
        fr3 = out3[blk - 1 : blk, :] - delta3
        curfix = jnp.maximum(
            (fr3.astype(jnp.bfloat16) + x_ref[blk - 1 : blk, :]) + bias3_ref[...],
            jnp.zeros((), jnp.bfloat16))
        fr0 = jnp.dot(curfix, a0_ref[...], preferred_element_type=jnp.float32)
        out_ref[blk - 1 : blk, :] = fr0.astype(jnp.bfloat16)
        wrong = out0[blk - 1 : blk, :]
        d0 = fr0 - wrong
        d1 = fr0 * fr0 - wrong * wrong
        so_ref[...] = so_ref[...] + jnp.concatenate([d0, d1], axis=0)


def _w2(Ws):
    if len(Ws) == 3:
        return ((Ws[0] - Ws[2]).astype(jnp.bfloat16),
                (-Ws[1]).astype(jnp.bfloat16),
                (2.0 * Ws[2]).astype(jnp.bfloat16))
    return Ws[0].astype(jnp.bfloat16), (-Ws[1]).astype(jnp.bfloat16), None


def _gn_inputs(stats, bprev, gnp, din):
    gamma, beta, alpha = gnp
    return [stats, bprev.reshape(1, din), gamma.reshape(1, din),
            beta.reshape(1, din), alpha.reshape(1, din)]


def _blkshape(n_total):
    blk = _BLK if n_total % _BLK == 0 else n_total
    return blk, n_total // blk


_CP = pltpu.CompilerParams(dimension_semantics=("arbitrary",))


def _run_stage(h, stats, bprev, gnp, Ws, bias, xres, lin, *, emit_stats,
               final, emit_xbf=False):
    n_total, din = h.shape
    dout = Ws[0].shape[1]
    k3 = len(Ws) == 3
    blk, nblk = _blkshape(n_total)
    wa, wb, wc = _w2(Ws)

    const = lambda shape: pl.BlockSpec(shape, lambda i: (0, 0))
    rows = lambda width: pl.BlockSpec((blk, width), lambda i: (i, 0))

    inputs = [h]
    in_specs = [rows(din)]
    gn = stats is not None
    if gn:
        inputs += _gn_inputs(stats, bprev, gnp, din)
        in_specs += [const((2, din))] + [const((1, din))] * 4
    inputs += [wa, wb] + ([wc] if k3 else [])
    in_specs += [const((din, dout))] * (3 if k3 else 2)
    resx = xres is not None
    if resx or final:
        inputs.append(bias.reshape(1, dout).astype(jnp.bfloat16))
        in_specs.append(const((1, dout)))
    if resx:
        inputs.append(xres)
        in_specs.append(rows(xres.shape[1]))
    scratch = [pltpu.VMEM((16, din), jnp.bfloat16)]
    if final:
        linw, linb = lin
        inputs += [linw, linb]
        in_specs += [const(linw.shape), const((1, linb.shape[-1]))]
        out_shape = jax.ShapeDtypeStruct((1, linb.shape[-1]), jnp.float32)
        out_specs = const((1, linb.shape[-1]))
        scratch.append(pltpu.VMEM((8, dout), jnp.float32))
    else:
        out_shape = [jax.ShapeDtypeStruct((n_total, dout), jnp.bfloat16)]
        out_specs = [rows(dout)]
        if emit_stats:
            out_shape.append(jax.ShapeDtypeStruct((2, dout), jnp.float32))
            out_specs.append(const((2, dout)))
        if emit_xbf:
            out_shape.append(jax.ShapeDtypeStruct((n_total, din), jnp.bfloat16))
            out_specs.append(rows(din))
        out_shape = tuple(out_shape)
        out_specs = tuple(out_specs)

    body = functools.partial(
        _stage_body, n_total=n_total, nblk=nblk, blk=blk, k3=k3, gn=gn,
        resx=resx, final=final, emit_stats=emit_stats, emit_xbf=emit_xbf)
    return pl.pallas_call(
        body, grid=(nblk,), in_specs=in_specs, out_specs=out_specs,
        out_shape=out_shape, scratch_shapes=scratch, compiler_params=_CP,
    )(*inputs)


def _run_merged(h, stats, bprev, gnp, Ws3, bias3, xres, Ws0):
    n_total, din = h.shape
    dmid = Ws3[0].shape[1]
    dout = Ws0[0].shape[1]
    blk, nblk = _blkshape(n_total)
    wa3, wb3, _ = _w2(Ws3)
    wa0, wb0, _ = _w2(Ws0)

    const = lambda shape: pl.BlockSpec(shape, lambda i: (0, 0))
    rows = lambda width: pl.BlockSpec((blk, width), lambda i: (i, 0))

    inputs = ([h] + _gn_inputs(stats, bprev, gnp, din) +
              [wa3, wb3, bias3.reshape(1, dmid).astype(jnp.bfloat16), xres,
               wa0, wb0])
    in_specs = ([rows(din), const((2, din))] + [const((1, din))] * 4 +
                [const((din, dmid))] * 2 + [const((1, dmid)), rows(dmid)] +
                [const((dmid, dout))] * 2)
    out_shape = (jax.ShapeDtypeStruct((n_total, dout), jnp.bfloat16),
                 jax.ShapeDtypeStruct((2, dout), jnp.float32))
    out_specs = (rows(dout), const((2, dout)))
    scratch = [pltpu.VMEM((16, din), jnp.bfloat16),
               pltpu.VMEM((16, dmid), jnp.bfloat16)]
    body = functools.partial(_merged_body, n_total=n_total, nblk=nblk, blk=blk)
    return pl.pallas_call(
        body, grid=(nblk,), in_specs=in_specs, out_specs=out_specs,
        out_shape=out_shape, scratch_shapes=scratch, compiler_params=_CP,
    )(*inputs)


def kernel(x, params):
    convs = params["convs"]
    gns = params["gns"]
    lin = (params["lin_W"].T, params["lin_b"].reshape(1, -1))

    h, stats, x_res = _run_stage(
        x, None, None, None, convs[0]["Ws"], convs[0]["b"], None, None,
        emit_stats=True, final=False, emit_xbf=True)

    for blk_i in range(4):
        for j in (1, 2):
            ci = 4 * blk_i + j
            g = gns[3 * blk_i + (j - 1)]
            gnp = (g["gamma"], g["beta"], g["alpha"])
            h, stats = _run_stage(
                h, stats, convs[ci - 1]["b"], gnp, convs[ci]["Ws"],
                convs[ci]["b"], None, None, emit_stats=True, final=False)
        ci = 4 * blk_i + 3
        g = gns[3 * blk_i + 2]
        gnp = (g["gamma"], g["beta"], g["alpha"])
        if blk_i < 3:
            cur = _run_stage(
                h, stats, convs[ci - 1]["b"], gnp, convs[ci]["Ws"],
                convs[ci]["b"], x_res, None, emit_stats=False, final=False)[0]
            h, stats = _run_stage(
                cur, None, None, None, convs[ci + 1]["Ws"],
                convs[ci + 1]["b"], None, None, emit_stats=True, final=False)
        else:
            return _run_stage(
                h, stats, convs[ci - 1]["b"], gnp, convs[ci]["Ws"],
                convs[ci]["b"], x_res, lin, emit_stats=False, final=True)
